# baseline (speedup 1.0000x reference)
import jax
import jax.numpy as jnp
from jax.experimental import pallas as pl
from jax.experimental.pallas import tpu as pltpu

ALPHA = 2.0
TT = 256


def _dot(a, b, dims=((1,), (0,))):
    return jax.lax.dot_general(a, b, (dims, ((), ())),
                               preferred_element_type=jnp.float32)


def _moe_kernel(x_ref, gw_ref, w1_ref, w2_ref, w3_ref,
                a1t_ref, b1r_ref, a2s_ref, b2_ref, a3t_ref, b3r_ref,
                o_ref):
    E, Dm = gw_ref.shape
    R = a1t_ref.shape[2]
    x = x_ref[:]

    logits_t = _dot(gw_ref[:], x, (((1,), (1,))))
    eidx = jax.lax.broadcasted_iota(jnp.int32, logits_t.shape, 0)
    m1 = jnp.max(logits_t, axis=0, keepdims=True)
    i1 = jnp.min(jnp.where(logits_t == m1, eidx, E), axis=0, keepdims=True)
    sel1 = eidx == i1
    masked = jnp.where(sel1, -jnp.inf, logits_t)
    m2 = jnp.max(masked, axis=0, keepdims=True)
    i2 = jnp.min(jnp.where(masked == m2, eidx, E), axis=0, keepdims=True)
    sel2 = eidx == i2
    p = jnp.exp(m2 - m1)
    inv = 1.0 / (1.0 + p)
    wgt = ((sel1.astype(jnp.float32) + sel2.astype(jnp.float32) * p) * inv).T

    base1 = _dot(x, w1_ref[:], (((1,), (1,))))
    base3 = _dot(x, w3_ref[:], (((1,), (1,))))
    xB1 = _dot(x, b1r_ref[:], (((1,), (1,))))
    xB3 = _dot(x, b3r_ref[:], (((1,), (1,))))

    hbar = jnp.zeros_like(base1)
    t2s = []
    for e in range(E):
        t1 = xB1[:, e * R:(e + 1) * R]
        t3 = xB3[:, e * R:(e + 1) * R]
        g = base1 + ALPHA * _dot(t1, a1t_ref[e], (((1,), (1,))))
        u = base3 + ALPHA * _dot(t3, a3t_ref[e], (((1,), (1,))))
        h = (g * jax.nn.sigmoid(g)) * u
        hw = wgt[:, e:e + 1] * h
        hbar = hbar + hw
        t2s.append(_dot(hw, b2_ref[e], (((1,), (1,)))))

    t2all = jnp.concatenate(t2s, axis=1)
    out = _dot(hbar, w2_ref[:], (((1,), (1,))))
    out = out + ALPHA * _dot(t2all, a2s_ref[:])
    o_ref[:] = out


@jax.jit
def kernel(inputs, gate_w, w1, w2, w3, A1, B1, A2, B2, A3, B3):
    T, D = inputs.shape
    H = w1.shape[0]
    E, _, R = A1.shape

    b1r = B1.reshape(E * R, H)
    a2s = A2.transpose(0, 2, 1).reshape(E * R, D)
    b3r = B3.reshape(E * R, D)

    whole = lambda shape: pl.BlockSpec(shape, lambda i: (0,) * len(shape))
    return pl.pallas_call(
        _moe_kernel,
        grid=(T // TT,),
        in_specs=[
            pl.BlockSpec((TT, D), lambda i: (i, 0)),
            whole((E, D)),
            whole((H, D)),
            whole((D, H)),
            whole((H, D)),
            whole((E, D, R)),
            whole((E * R, H)),
            whole((E * R, D)),
            whole((E, R, H)),
            whole((E, H, R)),
            whole((E * R, D)),
        ],
        out_specs=pl.BlockSpec((TT, D), lambda i: (i, 0)),
        out_shape=jax.ShapeDtypeStruct((T, D), jnp.float32),
        compiler_params=pltpu.CompilerParams(
            dimension_semantics=("arbitrary",),
        ),
    )(inputs, gate_w, w1, w2, w3, A1, b1r, a2s, B2, A3, b3r)


# speedup vs baseline: 1.1886x; 1.1886x over previous
import jax
import jax.numpy as jnp
from jax.experimental import pallas as pl
from jax.experimental.pallas import tpu as pltpu

ALPHA = 2.0
TT = 512


def _dot(a, b, dims=((1,), (0,))):
    return jax.lax.dot_general(a, b, (dims, ((), ())),
                               preferred_element_type=jnp.float32)


def _moe_kernel(x_ref, gw_ref, w1_ref, w2_ref, w3_ref,
                a1t_ref, b1r_ref, a2s_ref, b2_ref, a3t_ref, b3r_ref,
                o_ref):
    E, Dm = gw_ref.shape
    R = a1t_ref.shape[1]
    x = x_ref[:]

    logits_t = _dot(gw_ref[:], x, (((1,), (1,))))
    eidx = jax.lax.broadcasted_iota(jnp.int32, logits_t.shape, 0)
    m1 = jnp.max(logits_t, axis=0, keepdims=True)
    i1 = jnp.min(jnp.where(logits_t == m1, eidx, E), axis=0, keepdims=True)
    sel1 = eidx == i1
    masked = jnp.where(sel1, -jnp.inf, logits_t)
    m2 = jnp.max(masked, axis=0, keepdims=True)
    i2 = jnp.min(jnp.where(masked == m2, eidx, E), axis=0, keepdims=True)
    sel2 = eidx == i2
    p = jnp.exp(m2 - m1)
    inv = 1.0 / (1.0 + p)
    wgt = ((sel1.astype(jnp.float32) + sel2.astype(jnp.float32) * p) * inv).T

    base1 = _dot(x, w1_ref[:], (((1,), (1,))))
    base3 = _dot(x, w3_ref[:], (((1,), (1,))))
    xB1 = _dot(x, b1r_ref[:], (((1,), (1,))))
    xB3 = _dot(x, b3r_ref[:], (((1,), (1,))))

    hbar = jnp.zeros_like(base1)
    t2s = []
    for e in range(E):
        t1 = xB1[:, e * R:(e + 1) * R]
        t3 = xB3[:, e * R:(e + 1) * R]
        g = base1 + ALPHA * _dot(t1, a1t_ref[e])
        u = base3 + ALPHA * _dot(t3, a3t_ref[e])
        h = (g * jax.nn.sigmoid(g)) * u
        hw = wgt[:, e:e + 1] * h
        hbar = hbar + hw
        t2s.append(_dot(hw, b2_ref[e], (((1,), (1,)))))

    t2all = jnp.concatenate(t2s, axis=1)
    out = _dot(hbar, w2_ref[:], (((1,), (1,))))
    out = out + ALPHA * _dot(t2all, a2s_ref[:])
    o_ref[:] = out


@jax.jit
def kernel(inputs, gate_w, w1, w2, w3, A1, B1, A2, B2, A3, B3):
    T, D = inputs.shape
    H = w1.shape[0]
    E, _, R = A1.shape

    a1t = A1.transpose(0, 2, 1)
    a3t = A3.transpose(0, 2, 1)
    b1r = B1.reshape(E * R, H)
    a2s = A2.transpose(0, 2, 1).reshape(E * R, D)
    b3r = B3.reshape(E * R, D)

    whole = lambda shape: pl.BlockSpec(shape, lambda i: (0,) * len(shape))
    return pl.pallas_call(
        _moe_kernel,
        grid=(T // TT,),
        in_specs=[
            pl.BlockSpec((TT, D), lambda i: (i, 0)),
            whole((E, D)),
            whole((H, D)),
            whole((D, H)),
            whole((H, D)),
            whole((E, R, D)),
            whole((E * R, H)),
            whole((E * R, D)),
            whole((E, R, H)),
            whole((E, R, H)),
            whole((E * R, D)),
        ],
        out_specs=pl.BlockSpec((TT, D), lambda i: (i, 0)),
        out_shape=jax.ShapeDtypeStruct((T, D), jnp.float32),
        compiler_params=pltpu.CompilerParams(
            dimension_semantics=("arbitrary",),
            vmem_limit_bytes=120 * 1024 * 1024,
        ),
    )(inputs, gate_w, w1, w2, w3, a1t, b1r, a2s, B2, a3t, b3r)
